# Initial kernel scaffold; baseline (speedup 1.0000x reference)
import jax
import jax.numpy as jnp
from jax.experimental import pallas as pl


def kernel(pred, gt, mask):
    raise NotImplementedError("write your pallas kernel here")


# speedup vs baseline: 95.3128x; 95.3128x over previous
import jax
import jax.numpy as jnp
from jax.experimental import pallas as pl
from jax.experimental.pallas import tpu as pltpu

R = 800
C = 8192
BR = 80
GRID = R // BR
NEGATIVE_RATIO = 3.0
EPS = 1e-06


def _bce(x, z):
    return jnp.maximum(x, 0.0) - x * z + jnp.log1p(jnp.exp(-jnp.abs(x)))


def _stats_kernel(x_ref, z_ref, m_ref, out_ref):
    i = pl.program_id(0)

    @pl.when(i == 0)
    def _init():
        out_ref[0] = 0.0
        out_ref[1] = 0.0
        out_ref[2] = 0.0
        out_ref[3] = 0.0

    x = x_ref[...]
    z = z_ref[...]
    m = m_ref[...]
    pos = z * m
    neg = m - pos
    loss = _bce(x, z)
    out_ref[0] += jnp.sum(pos)
    out_ref[1] += jnp.sum(neg)
    out_ref[2] += jnp.sum(loss * pos)
    out_ref[3] += jnp.sum(loss * neg)


def _negloss_kernel(x_ref, z_ref, m_ref, out_ref):
    x = x_ref[...]
    z = z_ref[...]
    m = m_ref[...]
    out_ref[...] = _bce(x, z) * (m - z * m)


def _select_kernel(k_ref, v_ref, t_ref, s_ref, pfx_ref, acc_ref):
    p = pl.program_id(0)
    j = pl.program_id(1)

    @pl.when((p == 0) & (j == 0))
    def _init_prefix():
        pfx_ref[0] = 0

    @pl.when(j == 0)
    def _init_acc():
        acc_ref[0] = 0.0
        acc_ref[1] = 0.0

    v = v_ref[...]
    bits = jax.lax.bitcast_convert_type(v, jnp.int32)
    prefix = pfx_ref[0]

    @pl.when(p < 31)
    def _count_phase():
        cand = prefix | jnp.left_shift(jnp.int32(1), 30 - p)
        acc_ref[0] += jnp.sum((bits >= cand).astype(jnp.float32))

    @pl.when(p == 31)
    def _sum_phase():
        gt = bits > prefix
        acc_ref[0] += jnp.sum(gt.astype(jnp.float32))
        acc_ref[1] += jnp.sum(jnp.where(gt, v, 0.0))

    @pl.when((j == GRID - 1) & (p < 31))
    def _decide():
        cand = pfx_ref[0] | jnp.left_shift(jnp.int32(1), 30 - p)
        pfx_ref[0] = jnp.where(acc_ref[0] >= k_ref[0], cand, pfx_ref[0])

    @pl.when((j == GRID - 1) & (p == 31))
    def _emit():
        t_ref[0] = pfx_ref[0]
        s_ref[0] = acc_ref[0]
        s_ref[1] = acc_ref[1]


def _in_specs():
    return [pl.BlockSpec((BR, C), lambda i: (i, 0)) for _ in range(3)]


def _topk_sum_fallback(x, z, m, k):
    negloss = pl.pallas_call(
        _negloss_kernel,
        grid=(GRID,),
        in_specs=_in_specs(),
        out_specs=pl.BlockSpec((BR, C), lambda i: (i, 0)),
        out_shape=jax.ShapeDtypeStruct((R, C), jnp.float32),
    )(x, z, m)
    kf = jnp.reshape(k, (1,)).astype(jnp.float32)
    t, s = pl.pallas_call(
        _select_kernel,
        grid=(32, GRID),
        in_specs=[
            pl.BlockSpec(memory_space=pltpu.SMEM),
            pl.BlockSpec((BR, C), lambda p, j: (j, 0)),
        ],
        out_specs=[
            pl.BlockSpec(memory_space=pltpu.SMEM),
            pl.BlockSpec(memory_space=pltpu.SMEM),
        ],
        out_shape=[
            jax.ShapeDtypeStruct((1,), jnp.int32),
            jax.ShapeDtypeStruct((2,), jnp.float32),
        ],
        scratch_shapes=[
            pltpu.SMEM((1,), jnp.int32),
            pltpu.SMEM((2,), jnp.float32),
        ],
    )(kf, negloss)
    tval = jax.lax.bitcast_convert_type(t[0], jnp.float32)
    cnt_gt, sum_gt = s[0], s[1]
    return sum_gt + (k - cnt_gt) * tval


def kernel(pred, gt, mask):
    x = pred.reshape(R, C)
    z = gt.reshape(R, C)
    m = mask.reshape(R, C)
    stats = pl.pallas_call(
        _stats_kernel,
        grid=(GRID,),
        in_specs=_in_specs(),
        out_specs=pl.BlockSpec(memory_space=pltpu.SMEM),
        out_shape=jax.ShapeDtypeStruct((4,), jnp.float32),
    )(x, z, m)
    pos_cnt = jnp.floor(stats[0])
    neg_cnt = jnp.floor(stats[1])
    pos_sum = stats[2]
    neg_sum = stats[3]
    k = jnp.minimum(neg_cnt, jnp.floor(pos_cnt * NEGATIVE_RATIO))
    need_fallback = jnp.logical_and(k < neg_cnt, k > 0.0)
    topk = jax.lax.cond(
        need_fallback,
        lambda: _topk_sum_fallback(x, z, m, k),
        lambda: jnp.where(k >= neg_cnt, neg_sum, 0.0),
    )
    return (pos_sum + topk) / (pos_cnt + k + EPS)


# speedup vs baseline: 98.5865x; 1.0343x over previous
import jax
import jax.numpy as jnp
from jax.experimental import pallas as pl
from jax.experimental.pallas import tpu as pltpu

R = 800
C = 8192
BR = 80
GRID = R // BR
NEGATIVE_RATIO = 3.0
EPS = 1e-06


def _bce(x, z):
    return jnp.maximum(x, 0.0) - x * z + jnp.log1p(jnp.exp(-jnp.abs(x)))


def _stats_kernel(x_ref, z_ref, m_ref, out_ref):
    i = pl.program_id(0)

    @pl.when(i == 0)
    def _init():
        out_ref[0] = 0.0
        out_ref[1] = 0.0
        out_ref[2] = 0.0

    x = x_ref[...]
    z = z_ref[...]
    m = m_ref[...]
    pm = z * m
    sp = jnp.maximum(x, 0.0) + jnp.log1p(jnp.exp(-jnp.abs(x)))
    out_ref[0] += jnp.sum(pm)
    out_ref[1] += jnp.sum(m)
    out_ref[2] += jnp.sum(m * sp - x * pm)


def _negloss_kernel(x_ref, z_ref, m_ref, out_ref, ps_ref):
    i = pl.program_id(0)

    @pl.when(i == 0)
    def _init():
        ps_ref[0] = 0.0

    x = x_ref[...]
    z = z_ref[...]
    m = m_ref[...]
    loss = _bce(x, z)
    pm = z * m
    out_ref[...] = loss * (m - pm)
    ps_ref[0] += jnp.sum(loss * pm)


def _select_kernel(k_ref, v_ref, t_ref, s_ref, pfx_ref, acc_ref):
    p = pl.program_id(0)
    j = pl.program_id(1)

    @pl.when((p == 0) & (j == 0))
    def _init_prefix():
        pfx_ref[0] = 0

    @pl.when(j == 0)
    def _init_acc():
        acc_ref[0] = 0.0
        acc_ref[1] = 0.0

    v = v_ref[...]
    bits = jax.lax.bitcast_convert_type(v, jnp.int32)
    prefix = pfx_ref[0]

    @pl.when(p < 31)
    def _count_phase():
        cand = prefix | jnp.left_shift(jnp.int32(1), 30 - p)
        acc_ref[0] += jnp.sum((bits >= cand).astype(jnp.float32))

    @pl.when(p == 31)
    def _sum_phase():
        gt = bits > prefix
        acc_ref[0] += jnp.sum(gt.astype(jnp.float32))
        acc_ref[1] += jnp.sum(jnp.where(gt, v, 0.0))

    @pl.when((j == GRID - 1) & (p < 31))
    def _decide():
        cand = pfx_ref[0] | jnp.left_shift(jnp.int32(1), 30 - p)
        pfx_ref[0] = jnp.where(acc_ref[0] >= k_ref[0], cand, pfx_ref[0])

    @pl.when((j == GRID - 1) & (p == 31))
    def _emit():
        t_ref[0] = pfx_ref[0]
        s_ref[0] = acc_ref[0]
        s_ref[1] = acc_ref[1]


def _in_specs():
    return [pl.BlockSpec((BR, C), lambda i: (i, 0)) for _ in range(3)]


def _topk_numerator_fallback(x, z, m, k):
    negloss, pos_sum = pl.pallas_call(
        _negloss_kernel,
        grid=(GRID,),
        in_specs=_in_specs(),
        out_specs=[
            pl.BlockSpec((BR, C), lambda i: (i, 0)),
            pl.BlockSpec(memory_space=pltpu.SMEM),
        ],
        out_shape=[
            jax.ShapeDtypeStruct((R, C), jnp.float32),
            jax.ShapeDtypeStruct((1,), jnp.float32),
        ],
    )(x, z, m)
    kf = jnp.reshape(k, (1,)).astype(jnp.float32)
    t, s = pl.pallas_call(
        _select_kernel,
        grid=(32, GRID),
        in_specs=[
            pl.BlockSpec(memory_space=pltpu.SMEM),
            pl.BlockSpec((BR, C), lambda p, j: (j, 0)),
        ],
        out_specs=[
            pl.BlockSpec(memory_space=pltpu.SMEM),
            pl.BlockSpec(memory_space=pltpu.SMEM),
        ],
        out_shape=[
            jax.ShapeDtypeStruct((1,), jnp.int32),
            jax.ShapeDtypeStruct((2,), jnp.float32),
        ],
        scratch_shapes=[
            pltpu.SMEM((1,), jnp.int32),
            pltpu.SMEM((2,), jnp.float32),
        ],
    )(kf, negloss)
    tval = jax.lax.bitcast_convert_type(t[0], jnp.float32)
    cnt_gt, sum_gt = s[0], s[1]
    return pos_sum[0] + sum_gt + (k - cnt_gt) * tval


def kernel(pred, gt, mask):
    x = pred.reshape(R, C)
    z = gt.reshape(R, C)
    m = mask.reshape(R, C)
    stats = pl.pallas_call(
        _stats_kernel,
        grid=(GRID,),
        in_specs=_in_specs(),
        out_specs=pl.BlockSpec(memory_space=pltpu.SMEM),
        out_shape=jax.ShapeDtypeStruct((3,), jnp.float32),
    )(x, z, m)
    pos_cnt = jnp.floor(stats[0])
    neg_cnt = jnp.floor(stats[1] - stats[0])
    masked_sum = stats[2]
    k = jnp.minimum(neg_cnt, jnp.floor(pos_cnt * NEGATIVE_RATIO))
    need_fallback = jnp.logical_and(k < neg_cnt, k > 0.0)
    numerator = jax.lax.cond(
        need_fallback,
        lambda: _topk_numerator_fallback(x, z, m, k),
        lambda: jnp.where(k >= neg_cnt, masked_sum, 0.0),
    )
    return numerator / (pos_cnt + k + EPS)


# speedup vs baseline: 294.4246x; 2.9865x over previous
import jax
import jax.numpy as jnp
from jax.experimental import pallas as pl
from jax.experimental.pallas import tpu as pltpu

SHAPE = (16, 1, 640, 640)
BN = 2
GRID = SHAPE[0] // BN
BLK = (BN, 1, 640, 640)
NEGATIVE_RATIO = 3.0
EPS = 1e-06


def _bce(x, z):
    return jnp.maximum(x, 0.0) - x * z + jnp.log1p(jnp.exp(-jnp.abs(x)))


def _stats_kernel(x_ref, z_ref, m_ref, out_ref):
    i = pl.program_id(0)

    @pl.when(i == 0)
    def _init():
        out_ref[0] = 0.0
        out_ref[1] = 0.0
        out_ref[2] = 0.0

    x = x_ref[...]
    z = z_ref[...]
    m = m_ref[...]
    pm = z * m
    sp = jnp.maximum(x, 0.0) + jnp.log1p(jnp.exp(-jnp.abs(x)))
    out_ref[0] += jnp.sum(pm)
    out_ref[1] += jnp.sum(m)
    out_ref[2] += jnp.sum(m * sp - x * pm)


def _negloss_kernel(x_ref, z_ref, m_ref, out_ref, ps_ref):
    i = pl.program_id(0)

    @pl.when(i == 0)
    def _init():
        ps_ref[0] = 0.0

    x = x_ref[...]
    z = z_ref[...]
    m = m_ref[...]
    loss = _bce(x, z)
    pm = z * m
    out_ref[...] = loss * (m - pm)
    ps_ref[0] += jnp.sum(loss * pm)


def _select_kernel(k_ref, v_ref, t_ref, s_ref, pfx_ref, acc_ref):
    p = pl.program_id(0)
    j = pl.program_id(1)

    @pl.when((p == 0) & (j == 0))
    def _init_prefix():
        pfx_ref[0] = 0

    @pl.when(j == 0)
    def _init_acc():
        acc_ref[0] = 0.0
        acc_ref[1] = 0.0

    v = v_ref[...]
    bits = jax.lax.bitcast_convert_type(v, jnp.int32)
    prefix = pfx_ref[0]

    @pl.when(p < 31)
    def _count_phase():
        cand = prefix | jnp.left_shift(jnp.int32(1), 30 - p)
        acc_ref[0] += jnp.sum((bits >= cand).astype(jnp.float32))

    @pl.when(p == 31)
    def _sum_phase():
        gt = bits > prefix
        acc_ref[0] += jnp.sum(gt.astype(jnp.float32))
        acc_ref[1] += jnp.sum(jnp.where(gt, v, 0.0))

    @pl.when((j == GRID - 1) & (p < 31))
    def _decide():
        cand = pfx_ref[0] | jnp.left_shift(jnp.int32(1), 30 - p)
        pfx_ref[0] = jnp.where(acc_ref[0] >= k_ref[0], cand, pfx_ref[0])

    @pl.when((j == GRID - 1) & (p == 31))
    def _emit():
        t_ref[0] = pfx_ref[0]
        s_ref[0] = acc_ref[0]
        s_ref[1] = acc_ref[1]


def _in_specs():
    return [pl.BlockSpec(BLK, lambda i: (i, 0, 0, 0)) for _ in range(3)]


def _topk_numerator_fallback(x, z, m, k):
    negloss, pos_sum = pl.pallas_call(
        _negloss_kernel,
        grid=(GRID,),
        in_specs=_in_specs(),
        out_specs=[
            pl.BlockSpec(BLK, lambda i: (i, 0, 0, 0)),
            pl.BlockSpec(memory_space=pltpu.SMEM),
        ],
        out_shape=[
            jax.ShapeDtypeStruct(SHAPE, jnp.float32),
            jax.ShapeDtypeStruct((1,), jnp.float32),
        ],
    )(x, z, m)
    kf = jnp.reshape(k, (1,)).astype(jnp.float32)
    t, s = pl.pallas_call(
        _select_kernel,
        grid=(32, GRID),
        in_specs=[
            pl.BlockSpec(memory_space=pltpu.SMEM),
            pl.BlockSpec(BLK, lambda p, j: (j, 0, 0, 0)),
        ],
        out_specs=[
            pl.BlockSpec(memory_space=pltpu.SMEM),
            pl.BlockSpec(memory_space=pltpu.SMEM),
        ],
        out_shape=[
            jax.ShapeDtypeStruct((1,), jnp.int32),
            jax.ShapeDtypeStruct((2,), jnp.float32),
        ],
        scratch_shapes=[
            pltpu.SMEM((1,), jnp.int32),
            pltpu.SMEM((2,), jnp.float32),
        ],
    )(kf, negloss)
    tval = jax.lax.bitcast_convert_type(t[0], jnp.float32)
    cnt_gt, sum_gt = s[0], s[1]
    return pos_sum[0] + sum_gt + (k - cnt_gt) * tval


def kernel(pred, gt, mask):
    x, z, m = pred, gt, mask
    stats = pl.pallas_call(
        _stats_kernel,
        grid=(GRID,),
        in_specs=_in_specs(),
        out_specs=pl.BlockSpec(memory_space=pltpu.SMEM),
        out_shape=jax.ShapeDtypeStruct((3,), jnp.float32),
    )(x, z, m)
    pos_cnt = jnp.floor(stats[0])
    neg_cnt = jnp.floor(stats[1] - stats[0])
    masked_sum = stats[2]
    k = jnp.minimum(neg_cnt, jnp.floor(pos_cnt * NEGATIVE_RATIO))
    need_fallback = jnp.logical_and(k < neg_cnt, k > 0.0)
    numerator = jax.lax.cond(
        need_fallback,
        lambda: _topk_numerator_fallback(x, z, m, k),
        lambda: jnp.where(k >= neg_cnt, masked_sum, 0.0),
    )
    return numerator / (pos_cnt + k + EPS)
